# baseline (speedup 1.0000x reference)
import functools

import jax
import jax.numpy as jnp
from jax import lax
from jax.experimental import pallas as pl
from jax.experimental.pallas import tpu as pltpu
from jax.experimental.pallas import tpu_sc as plsc

N = 10000
C = 128
E = 320000
K = 32
NC = 2
NS = 16
NW = NC * NS
ZONE = 624
LAST_ZONE = N - (NS - 1) * ZONE
NBUF = 4
DIST = 3
CHUNKS = 320
CPR = 128 // K
EP = CHUNKS * K
E_PAD = EP * NW
RI = EP // 128

_SPLAT_DNUMS = lax.GatherDimensionNumbers(
    offset_dims=(), collapsed_slice_dims=(0,), start_index_map=(0,))


def _splat(v16, lane):
    idx = jnp.full((16, 1), lane, jnp.int32)
    return lax.gather(v16, idx, _SPLAT_DNUMS, slice_sizes=(1,),
                      mode=lax.GatherScatterMode.PROMISE_IN_BOUNDS)


def _make_sc_kernel():
    mesh = plsc.VectorSubcoreMesh(core_axis_name="c", subcore_axis_name="s")

    @functools.partial(
        pl.kernel,
        mesh=mesh,
        out_type=jax.ShapeDtypeStruct((NC, N, C), jnp.float32),
        scratch_types=(
            [pltpu.VMEM((RI, 128), jnp.int32),
             pltpu.VMEM((RI, 128), jnp.int32),
             pltpu.VMEM((RI, 128), jnp.float32),
             pltpu.VMEM_SHARED((N, C), jnp.float32)]
            + [pltpu.VMEM((K, C), jnp.float32)] * NBUF
            + [pltpu.SemaphoreType.DMA] * (2 * NBUF)
        ),
    )
    def sc_kernel(x_hbm, i_hbm, j_hbm, ew_hbm, out_hbm,
                  i_all, j_all, ew_all, acc, *bufs_sems):
        rows = bufs_sems[:NBUF]
        gsem = bufs_sems[NBUF:2 * NBUF]
        ssem = bufs_sems[2 * NBUF:]

        cid = lax.axis_index("c")
        sid = lax.axis_index("s")
        wid = cid * NS + sid

        with jax.named_scope("preload"):
            pltpu.sync_copy(i_hbm.at[wid], i_all)
            pltpu.sync_copy(j_hbm.at[wid], j_all)
            pltpu.sync_copy(ew_hbm.at[wid], ew_all)

        zero16 = jnp.zeros((16,), jnp.float32)

        def zero_row(r, carry):
            for cb in range(C // 16):
                rows[0][r, pl.ds(cb * 16, 16)] = zero16
            return carry

        with jax.named_scope("zerobuf"):
            lax.fori_loop(0, K, zero_row, 0)
        zone_base = sid * ZONE
        nfull = ZONE // K
        rem = ZONE % K
        assert LAST_ZONE == (nfull + 1) * K
        for zoff in range(0, nfull * K, K):
            pltpu.sync_copy(rows[0].at[pl.ds(0, K)],
                            acc.at[pl.ds(zone_base + zoff, K)])

        @pl.when(sid < NS - 1)
        def _():
            pltpu.sync_copy(rows[0].at[pl.ds(0, rem)],
                            acc.at[pl.ds(zone_base + nfull * K, rem)])

        @pl.when(sid == NS - 1)
        def _():
            pltpu.sync_copy(rows[0].at[pl.ds(0, K)],
                            acc.at[pl.ds(zone_base + nfull * K, K)])

        for d in range(DIST):
            pltpu.async_copy(x_hbm.at[i_all.at[0, pl.ds(d * K, K)]],
                             rows[d], gsem[d])

        plsc.subcore_barrier()

        dummy = x_hbm.at[pl.ds(0, K)]

        def scale_chunk(src, dst, row, off):
            for g in range(K // 16):
                w16 = ew_all[row, pl.ds(off + g * 16, 16)]
                ws = [_splat(w16, l) for l in range(16)]
                for cb in range(C // 16):
                    sl = pl.ds(cb * 16, 16)
                    for l in range(16):
                        r = g * 16 + l
                        dst[r, sl] = src[r, sl] * ws[l]

        def group_body(t, carry):
            for bi in range(NBUF):
                ck = t * NBUF + bi
                bp = (bi + DIST) % NBUF
                cp = ck + DIST

                @pl.when((cp < CHUNKS) & (cp >= NBUF))
                def _():
                    pltpu.make_async_copy(dummy, rows[bp], ssem[bp]).wait()

                @pl.when(cp < CHUNKS)
                def _():
                    idx = i_all.at[cp // CPR, pl.ds((cp % CPR) * K, K)]
                    pltpu.async_copy(x_hbm.at[idx], rows[bp], gsem[bp])

                pltpu.make_async_copy(dummy, rows[bi], gsem[bi]).wait()
                scale_chunk(rows[bi], rows[bi], ck // CPR, (ck % CPR) * K)
                pltpu.async_copy(
                    rows[bi],
                    acc.at[j_all.at[ck // CPR, pl.ds((ck % CPR) * K, K)]],
                    ssem[bi], add=True)
            return carry

        with jax.named_scope("mainloop"):
            lax.fori_loop(0, CHUNKS // NBUF, group_body, 0)
            for bi in range(NBUF):
                pltpu.make_async_copy(dummy, rows[bi], ssem[bi]).wait()
        plsc.subcore_barrier()

        with jax.named_scope("writeout"):
            @pl.when(sid < NS - 1)
            def _():
                pltpu.sync_copy(acc.at[pl.ds(zone_base, ZONE)],
                                out_hbm.at[cid, pl.ds(zone_base, ZONE)])

            @pl.when(sid == NS - 1)
            def _():
                pltpu.sync_copy(acc.at[pl.ds(zone_base, LAST_ZONE)],
                                out_hbm.at[cid, pl.ds(zone_base, LAST_ZONE)])

    return sc_kernel


def _combine_partials(parts):
    def add_body(p_ref, o_ref):
        o_ref[...] = p_ref[0] + p_ref[1]

    return pl.pallas_call(
        add_body,
        out_shape=jax.ShapeDtypeStruct((N, C), jnp.float32),
    )(parts)


@jax.jit
def kernel(x, g, ew):
    i = g[0].astype(jnp.int32)
    j = g[1].astype(jnp.int32)
    ew = ew.astype(jnp.float32)
    pad = E_PAD - E
    if pad:
        fill = (jnp.arange(pad, dtype=jnp.int32) * 13) % N
        i = jnp.concatenate([i, fill])
        j = jnp.concatenate([j, fill])
        ew = jnp.pad(ew, (0, pad))
    i = i.reshape(NW, RI, 128)
    j = j.reshape(NW, RI, 128)
    ew = ew.reshape(NW, RI, 128)
    parts = _make_sc_kernel()(x, i, j, ew)
    return _combine_partials(parts)


# speedup vs baseline: 1.0359x; 1.0359x over previous
import functools

import jax
import jax.numpy as jnp
from jax import lax
from jax.experimental import pallas as pl
from jax.experimental.pallas import tpu as pltpu
from jax.experimental.pallas import tpu_sc as plsc

N = 10000
C = 128
E = 320000
K = 16
NC = 2
NS = 16
NW = NC * NS
ZONE = 624
LAST_ZONE = N - (NS - 1) * ZONE
NBUF = 8
DIST = 4
CHUNKS = 640
CPR = 128 // K
EP = CHUNKS * K
E_PAD = EP * NW
RI = EP // 128

_SPLAT_DNUMS = lax.GatherDimensionNumbers(
    offset_dims=(), collapsed_slice_dims=(0,), start_index_map=(0,))


def _splat(v16, lane):
    idx = jnp.full((16, 1), lane, jnp.int32)
    return lax.gather(v16, idx, _SPLAT_DNUMS, slice_sizes=(1,),
                      mode=lax.GatherScatterMode.PROMISE_IN_BOUNDS)


def _make_sc_kernel():
    mesh = plsc.VectorSubcoreMesh(core_axis_name="c", subcore_axis_name="s")

    @functools.partial(
        pl.kernel,
        mesh=mesh,
        out_type=jax.ShapeDtypeStruct((NC, N, C), jnp.float32),
        scratch_types=(
            [pltpu.VMEM((RI, 128), jnp.int32),
             pltpu.VMEM((RI, 128), jnp.int32),
             pltpu.VMEM((RI, 128), jnp.float32),
             pltpu.VMEM_SHARED((N, C), jnp.float32)]
            + [pltpu.VMEM((K, C), jnp.float32)] * NBUF
            + [pltpu.SemaphoreType.DMA] * (2 * NBUF)
        ),
    )
    def sc_kernel(x_hbm, i_hbm, j_hbm, ew_hbm, out_hbm,
                  i_all, j_all, ew_all, acc, *bufs_sems):
        rows = bufs_sems[:NBUF]
        gsem = bufs_sems[NBUF:2 * NBUF]
        ssem = bufs_sems[2 * NBUF:]

        cid = lax.axis_index("c")
        sid = lax.axis_index("s")
        wid = cid * NS + sid

        with jax.named_scope("preload"):
            pltpu.sync_copy(i_hbm.at[wid], i_all)
            pltpu.sync_copy(j_hbm.at[wid], j_all)
            pltpu.sync_copy(ew_hbm.at[wid], ew_all)

        zero16 = jnp.zeros((16,), jnp.float32)

        def zero_row(r, carry):
            for cb in range(C // 16):
                rows[0][r, pl.ds(cb * 16, 16)] = zero16
            return carry

        with jax.named_scope("zerobuf"):
            lax.fori_loop(0, K, zero_row, 0)
        zone_base = sid * ZONE
        nfull = ZONE // K
        rem = ZONE % K
        assert LAST_ZONE == (nfull + 1) * K
        for zoff in range(0, nfull * K, K):
            pltpu.sync_copy(rows[0].at[pl.ds(0, K)],
                            acc.at[pl.ds(zone_base + zoff, K)])

        if rem:
            @pl.when(sid < NS - 1)
            def _():
                pltpu.sync_copy(rows[0].at[pl.ds(0, rem)],
                                acc.at[pl.ds(zone_base + nfull * K, rem)])

        @pl.when(sid == NS - 1)
        def _():
            pltpu.sync_copy(rows[0].at[pl.ds(0, K)],
                            acc.at[pl.ds(zone_base + nfull * K, K)])

        for d in range(DIST):
            pltpu.async_copy(x_hbm.at[i_all.at[0, pl.ds(d * K, K)]],
                             rows[d], gsem[d])

        plsc.subcore_barrier()

        dummy = x_hbm.at[pl.ds(0, K)]

        def scale_chunk(src, dst, row, off):
            for g in range(K // 16):
                w16 = ew_all[row, pl.ds(off + g * 16, 16)]
                ws = [_splat(w16, l) for l in range(16)]
                for cb in range(C // 16):
                    sl = pl.ds(cb * 16, 16)
                    for l in range(16):
                        r = g * 16 + l
                        dst[r, sl] = src[r, sl] * ws[l]

        def group_body(t, carry):
            for bi in range(NBUF):
                ck = t * NBUF + bi
                bp = (bi + DIST) % NBUF
                cp = ck + DIST

                @pl.when((cp < CHUNKS) & (cp >= NBUF))
                def _():
                    pltpu.make_async_copy(dummy, rows[bp], ssem[bp]).wait()

                @pl.when(cp < CHUNKS)
                def _():
                    idx = i_all.at[cp // CPR, pl.ds((cp % CPR) * K, K)]
                    pltpu.async_copy(x_hbm.at[idx], rows[bp], gsem[bp])

                pltpu.make_async_copy(dummy, rows[bi], gsem[bi]).wait()
                scale_chunk(rows[bi], rows[bi], ck // CPR, (ck % CPR) * K)
                pltpu.async_copy(
                    rows[bi],
                    acc.at[j_all.at[ck // CPR, pl.ds((ck % CPR) * K, K)]],
                    ssem[bi], add=True)
            return carry

        with jax.named_scope("mainloop"):
            lax.fori_loop(0, CHUNKS // NBUF, group_body, 0)
            for bi in range(NBUF):
                pltpu.make_async_copy(dummy, rows[bi], ssem[bi]).wait()
        plsc.subcore_barrier()

        with jax.named_scope("writeout"):
            @pl.when(sid < NS - 1)
            def _():
                pltpu.sync_copy(acc.at[pl.ds(zone_base, ZONE)],
                                out_hbm.at[cid, pl.ds(zone_base, ZONE)])

            @pl.when(sid == NS - 1)
            def _():
                pltpu.sync_copy(acc.at[pl.ds(zone_base, LAST_ZONE)],
                                out_hbm.at[cid, pl.ds(zone_base, LAST_ZONE)])

    return sc_kernel


def _combine_partials(parts):
    def add_body(p_ref, o_ref):
        o_ref[...] = p_ref[0] + p_ref[1]

    return pl.pallas_call(
        add_body,
        out_shape=jax.ShapeDtypeStruct((N, C), jnp.float32),
    )(parts)


@jax.jit
def kernel(x, g, ew):
    i = g[0].astype(jnp.int32)
    j = g[1].astype(jnp.int32)
    ew = ew.astype(jnp.float32)
    pad = E_PAD - E
    if pad:
        fill = (jnp.arange(pad, dtype=jnp.int32) * 13) % N
        i = jnp.concatenate([i, fill])
        j = jnp.concatenate([j, fill])
        ew = jnp.pad(ew, (0, pad))
    i = i.reshape(NW, RI, 128)
    j = j.reshape(NW, RI, 128)
    ew = ew.reshape(NW, RI, 128)
    parts = _make_sc_kernel()(x, i, j, ew)
    return _combine_partials(parts)
